# Initial kernel scaffold; baseline (speedup 1.0000x reference)
import jax
import jax.numpy as jnp
from jax.experimental import pallas as pl


def kernel(x, edge_index, batch, W1a, b1a, W1b, b1b, W2a, b2a, W2b, b2b, W3a, b3a, W3b, b3b, Wl, bl):
    raise NotImplementedError("write your pallas kernel here")


# speedup vs baseline: 4.1297x; 4.1297x over previous
import functools

import jax
import jax.numpy as jnp
from jax import lax
from jax.experimental import pallas as pl
from jax.experimental.pallas import tpu as pltpu
from jax.experimental.pallas import tpu_sc as plsc

N = 10000
E = 320000
G = 64
H = 256
OUT = 120

_NSUB = 16
_K = 128
_NCHUNK = E // _K
_ITERS = -(-_NCHUNK // _NSUB)
_ROW_CHUNK = 80
_NROWCHUNK = N // _ROW_CHUNK
_ROW_ITERS = -(-_NROWCHUNK // _NSUB)


def _make_agg1():
    mesh = plsc.VectorSubcoreMesh(core_axis_name="c", subcore_axis_name="s")
    half = _NCHUNK // 2
    iters = -(-half // _NSUB)

    @functools.partial(
        pl.kernel,
        mesh=mesh,
        out_type=[
            jax.ShapeDtypeStruct((N, 128), jnp.float32),
            jax.ShapeDtypeStruct((N, 128), jnp.float32),
        ],
        scratch_types=[
            pltpu.VMEM_SHARED((N, 128), jnp.float32),
            pltpu.VMEM((_K,), jnp.int32),
            pltpu.VMEM((_K,), jnp.int32),
            pltpu.VMEM((_K, 128), jnp.float32),
            pltpu.VMEM((_ROW_CHUNK, 128), jnp.float32),
            pltpu.SemaphoreType.DMA,
        ],
    )
    def agg1(x_hbm, zeros_hbm, src_hbm, dst_hbm, za, zb,
             acc, src_v, dst_v, rows_v, stage_v, sem):
        c = lax.axis_index("c")
        s = lax.axis_index("s")

        def init_from(h_ref):
            for t in range(_ROW_ITERS):
                j = t * _NSUB + s

                @pl.when(j < _NROWCHUNK)
                def _():
                    sl = pl.ds(j * _ROW_CHUNK, _ROW_CHUNK)
                    pltpu.sync_copy(h_ref.at[sl], stage_v)
                    pltpu.sync_copy(stage_v, acc.at[sl])

        def write_to(out_ref):
            for t in range(_ROW_ITERS):
                j = t * _NSUB + s

                @pl.when(j < _NROWCHUNK)
                def _():
                    sl = pl.ds(j * _ROW_CHUNK, _ROW_CHUNK)
                    pltpu.sync_copy(acc.at[sl], stage_v)
                    pltpu.sync_copy(stage_v, out_ref.at[sl])

        @pl.when(c == 0)
        def _():
            init_from(x_hbm)

        @pl.when(c == 1)
        def _():
            init_from(zeros_hbm)

        plsc.subcore_barrier()

        def body(i, carry):
            lid = i * _NSUB + s

            @pl.when(lid < half)
            def _():
                ebase = (c * half + lid) * _K
                pltpu.sync_copy(src_hbm.at[pl.ds(ebase, _K)], src_v)
                pltpu.sync_copy(dst_hbm.at[pl.ds(ebase, _K)], dst_v)
                pltpu.async_copy(x_hbm.at[src_v], rows_v, sem).wait()
                pltpu.sync_copy(rows_v, acc.at[dst_v], add=True)

            return carry

        lax.fori_loop(0, iters, body, 0)
        plsc.subcore_barrier()

        @pl.when(c == 0)
        def _():
            write_to(za)

        @pl.when(c == 1)
        def _():
            write_to(zb)

    return agg1


def _make_agg(d):
    mesh = plsc.VectorSubcoreMesh(core_axis_name="c", subcore_axis_name="s")

    @functools.partial(
        pl.kernel,
        mesh=mesh,
        out_type=[
            jax.ShapeDtypeStruct((N, d), jnp.float32),
            jax.ShapeDtypeStruct((N, d), jnp.float32),
        ],
        scratch_types=[
            pltpu.VMEM_SHARED((N, d), jnp.float32),
            pltpu.VMEM((_K,), jnp.int32),
            pltpu.VMEM((_K,), jnp.int32),
            pltpu.VMEM((_K, d), jnp.float32),
            pltpu.VMEM((_ROW_CHUNK, d), jnp.float32),
            pltpu.SemaphoreType.DMA,
        ],
    )
    def agg(h_lo, h_hi, src_hbm, dst_hbm, z_lo, z_hi,
            acc, src_v, dst_v, rows_v, stage_v, sem):
        c = lax.axis_index("c")
        s = lax.axis_index("s")

        def init_from(h_ref):
            for t in range(_ROW_ITERS):
                j = t * _NSUB + s

                @pl.when(j < _NROWCHUNK)
                def _():
                    sl = pl.ds(j * _ROW_CHUNK, _ROW_CHUNK)
                    pltpu.sync_copy(h_ref.at[sl], stage_v)
                    pltpu.sync_copy(stage_v, acc.at[sl])

        def edges_from(h_ref):
            def body(i, carry):
                cid = i * _NSUB + s

                @pl.when(cid < _NCHUNK)
                def _():
                    ebase = cid * _K
                    pltpu.sync_copy(src_hbm.at[pl.ds(ebase, _K)], src_v)
                    pltpu.sync_copy(dst_hbm.at[pl.ds(ebase, _K)], dst_v)
                    pltpu.async_copy(h_ref.at[src_v], rows_v, sem).wait()
                    pltpu.sync_copy(rows_v, acc.at[dst_v], add=True)

                return carry

            lax.fori_loop(0, _ITERS, body, 0)

        def write_to(out_ref):
            for t in range(_ROW_ITERS):
                j = t * _NSUB + s

                @pl.when(j < _NROWCHUNK)
                def _():
                    sl = pl.ds(j * _ROW_CHUNK, _ROW_CHUNK)
                    pltpu.sync_copy(acc.at[sl], stage_v)
                    pltpu.sync_copy(stage_v, out_ref.at[sl])

        @pl.when(c == 0)
        def _():
            init_from(h_lo)

        @pl.when(c == 1)
        def _():
            init_from(h_hi)

        plsc.subcore_barrier()

        @pl.when(c == 0)
        def _():
            edges_from(h_lo)

        @pl.when(c == 1)
        def _():
            edges_from(h_hi)

        plsc.subcore_barrier()

        @pl.when(c == 0)
        def _():
            write_to(z_lo)

        @pl.when(c == 1)
        def _():
            write_to(z_hi)

    return agg


_agg1 = _make_agg1()
_agg128 = _make_agg(128)

_R = 1000
_NB = N // _R

_PREC = lax.Precision.HIGHEST


def _mlp_sum_body(za_ref, zb_ref, wa_ref, ba_ref, wb_ref, bb_ref,
                  hlo_ref, hhi_ref):
    z = za_ref[...] + zb_ref[...]
    t = jax.nn.relu(jnp.dot(z, wa_ref[...],
                            preferred_element_type=jnp.float32,
                            precision=_PREC) + ba_ref[...])
    h = jax.nn.relu(jnp.dot(t, wb_ref[...],
                            preferred_element_type=jnp.float32,
                            precision=_PREC) + bb_ref[...])
    hlo_ref[...] = h[:, 0:H // 2]
    hhi_ref[...] = h[:, H // 2:H]


def _mlp_sum(za, zb, Wa, ba, Wb, bb):
    din = za.shape[1]
    return pl.pallas_call(
        _mlp_sum_body,
        grid=(_NB,),
        in_specs=[
            pl.BlockSpec((_R, din), lambda i: (i, 0)),
            pl.BlockSpec((_R, din), lambda i: (i, 0)),
            pl.BlockSpec(Wa.shape, lambda i: (0, 0)),
            pl.BlockSpec((1, H), lambda i: (0, 0)),
            pl.BlockSpec(Wb.shape, lambda i: (0, 0)),
            pl.BlockSpec((1, H), lambda i: (0, 0)),
        ],
        out_specs=[
            pl.BlockSpec((_R, H // 2), lambda i: (i, 0)),
            pl.BlockSpec((_R, H // 2), lambda i: (i, 0)),
        ],
        out_shape=[
            jax.ShapeDtypeStruct((N, H // 2), jnp.float32),
            jax.ShapeDtypeStruct((N, H // 2), jnp.float32),
        ],
    )(za, zb, Wa, ba.reshape(1, H), Wb, bb.reshape(1, H))


def _mlp_body(zlo_ref, zhi_ref, wa_ref, ba_ref, wb_ref, bb_ref,
              hlo_ref, hhi_ref):
    dh = zlo_ref.shape[1]
    z1 = jnp.dot(zlo_ref[...], wa_ref[0:dh, :],
                 preferred_element_type=jnp.float32, precision=_PREC)
    z2 = jnp.dot(zhi_ref[...], wa_ref[dh:2 * dh, :],
                 preferred_element_type=jnp.float32, precision=_PREC)
    t = jax.nn.relu(z1 + z2 + ba_ref[...])
    h = jax.nn.relu(jnp.dot(t, wb_ref[...],
                            preferred_element_type=jnp.float32,
                            precision=_PREC) + bb_ref[...])
    hlo_ref[...] = h[:, 0:H // 2]
    hhi_ref[...] = h[:, H // 2:H]


def _mlp(z_lo, z_hi, Wa, ba, Wb, bb):
    dh = z_lo.shape[1]
    return pl.pallas_call(
        _mlp_body,
        grid=(_NB,),
        in_specs=[
            pl.BlockSpec((_R, dh), lambda i: (i, 0)),
            pl.BlockSpec((_R, dh), lambda i: (i, 0)),
            pl.BlockSpec(Wa.shape, lambda i: (0, 0)),
            pl.BlockSpec((1, H), lambda i: (0, 0)),
            pl.BlockSpec(Wb.shape, lambda i: (0, 0)),
            pl.BlockSpec((1, H), lambda i: (0, 0)),
        ],
        out_specs=[
            pl.BlockSpec((_R, H // 2), lambda i: (i, 0)),
            pl.BlockSpec((_R, H // 2), lambda i: (i, 0)),
        ],
        out_shape=[
            jax.ShapeDtypeStruct((N, H // 2), jnp.float32),
            jax.ShapeDtypeStruct((N, H // 2), jnp.float32),
        ],
    )(z_lo, z_hi, Wa, ba.reshape(1, H), Wb, bb.reshape(1, H))


def _mlp_pool_body(zlo_ref, zhi_ref, wa_ref, ba_ref, wb_ref, bb_ref,
                   bat_ref, wl_ref, bl_ref, out_ref, pooled):
    i = pl.program_id(0)
    dh = zlo_ref.shape[1]
    z1 = jnp.dot(zlo_ref[...], wa_ref[0:dh, :],
                 preferred_element_type=jnp.float32, precision=_PREC)
    z2 = jnp.dot(zhi_ref[...], wa_ref[dh:2 * dh, :],
                 preferred_element_type=jnp.float32, precision=_PREC)
    t = jax.nn.relu(z1 + z2 + ba_ref[...])
    h = jax.nn.relu(jnp.dot(t, wb_ref[...],
                            preferred_element_type=jnp.float32,
                            precision=_PREC) + bb_ref[...])
    bat = bat_ref[0, 0, :]
    gid = lax.broadcasted_iota(jnp.int32, (G, _R), 0)
    sel = (bat[None, :] == gid).astype(jnp.float32)
    contrib = jnp.dot(sel, h, preferred_element_type=jnp.float32,
                      precision=_PREC)

    @pl.when(i == 0)
    def _():
        pooled[...] = contrib

    @pl.when(i > 0)
    def _():
        pooled[...] = pooled[...] + contrib

    @pl.when(i == _NB - 1)
    def _():
        out_ref[...] = jnp.dot(pooled[...], wl_ref[...],
                               preferred_element_type=jnp.float32,
                               precision=_PREC) + bl_ref[...]


def _mlp_pool(z_lo, z_hi, Wa, ba, Wb, bb, batch3, Wl, bl):
    dh = z_lo.shape[1]
    return pl.pallas_call(
        _mlp_pool_body,
        grid=(_NB,),
        in_specs=[
            pl.BlockSpec((_R, dh), lambda i: (i, 0)),
            pl.BlockSpec((_R, dh), lambda i: (i, 0)),
            pl.BlockSpec(Wa.shape, lambda i: (0, 0)),
            pl.BlockSpec((1, H), lambda i: (0, 0)),
            pl.BlockSpec(Wb.shape, lambda i: (0, 0)),
            pl.BlockSpec((1, H), lambda i: (0, 0)),
            pl.BlockSpec((1, 1, _R), lambda i: (i, 0, 0)),
            pl.BlockSpec(Wl.shape, lambda i: (0, 0)),
            pl.BlockSpec((1, OUT), lambda i: (0, 0)),
        ],
        out_specs=pl.BlockSpec((G, OUT), lambda i: (0, 0)),
        out_shape=jax.ShapeDtypeStruct((G, OUT), jnp.float32),
        scratch_shapes=[pltpu.VMEM((G, H), jnp.float32)],
    )(z_lo, z_hi, Wa, ba.reshape(1, H), Wb, bb.reshape(1, H),
      batch3, Wl, bl.reshape(1, OUT))


def kernel(x, edge_index, batch,
           W1a, b1a, W1b, b1b,
           W2a, b2a, W2b, b2b,
           W3a, b3a, W3b, b3b,
           Wl, bl):
    src = edge_index[0].astype(jnp.int32)
    dst = edge_index[1].astype(jnp.int32)
    batch3 = batch.astype(jnp.int32).reshape(_NB, 1, _R)

    zeros = jnp.zeros_like(x)

    z1_a, z1_b = _agg1(x, zeros, src, dst)
    h1_lo, h1_hi = _mlp_sum(z1_a, z1_b, W1a, b1a, W1b, b1b)
    z2_lo, z2_hi = _agg128(h1_lo, h1_hi, src, dst)
    h2_lo, h2_hi = _mlp(z2_lo, z2_hi, W2a, b2a, W2b, b2b)
    z3_lo, z3_hi = _agg128(h2_lo, h2_hi, src, dst)
    out = _mlp_pool(z3_lo, z3_hi, W3a, b3a, W3b, b3b, batch3, Wl, bl)
    return out
